# baseline (speedup 1.0000x reference)
import functools

import jax
import jax.numpy as jnp
from jax import lax
from jax.experimental import pallas as pl
from jax.experimental.pallas import tpu as pltpu
from jax.experimental.pallas import tpu_sc as plsc

B = 1024
S = 200
E = 64
R = B * S

_info = plsc.get_sparse_core_info()
NC, NS = _info.num_cores, _info.num_subcores
NW = NC * NS
R_PER_W = R // NW
IDX_W = 128
K = 2
CHUNK = IDX_W * K
N_CHUNKS = R_PER_W // CHUNK
IDX_ROWS = R_PER_W // IDX_W

_sc_mesh = plsc.VectorSubcoreMesh(core_axis_name="c", subcore_axis_name="s")


@functools.partial(
    pl.kernel,
    mesh=_sc_mesh,
    out_type=jax.ShapeDtypeStruct((R, 2 * E), jnp.float32),
    scratch_types=[
        pltpu.VMEM((IDX_ROWS, IDX_W), jnp.int32),
        pltpu.VMEM((IDX_ROWS, IDX_W), jnp.int32),
        pltpu.VMEM((2, CHUNK, E), jnp.float32),
        pltpu.VMEM((2, CHUNK, E), jnp.float32),
        pltpu.SemaphoreType.DMA,
        pltpu.SemaphoreType.DMA,
    ],
    compiler_params=pltpu.CompilerParams(use_tc_tiling_on_sc=False),
)
def _sc_gather(table_hbm, ps_hbm, idx_hbm, cidx_hbm, out_hbm,
               idx_v, cidx_v, rows_v, ps_v, sem, semo):
    wid = lax.axis_index("s") * NC + lax.axis_index("c")
    base = wid * R_PER_W
    pltpu.sync_copy(idx_hbm.at[wid], idx_v)
    pltpu.sync_copy(cidx_hbm.at[wid], cidx_v)

    def fire(c, b):
        for j in range(K):
            pltpu.async_copy(
                table_hbm.at[idx_v.at[c * K + j]],
                rows_v.at[b].at[pl.ds(j * IDX_W, IDX_W)],
                sem,
            )
            pltpu.async_copy(
                ps_hbm.at[cidx_v.at[c * K + j]],
                ps_v.at[b].at[pl.ds(j * IDX_W, IDX_W)],
                sem,
            )

    def drain_gathers(b):
        pltpu.make_async_copy(table_hbm.at[idx_v.at[0]], rows_v.at[b], sem).wait()
        pltpu.make_async_copy(ps_hbm.at[cidx_v.at[0]], ps_v.at[b], sem).wait()

    def out_async(c, b):
        pltpu.make_async_copy(
            rows_v.at[b],
            out_hbm.at[pl.ds(base + c * CHUNK, CHUNK), pl.ds(0, E)],
            semo,
        ).start()
        pltpu.make_async_copy(
            ps_v.at[b],
            out_hbm.at[pl.ds(base + c * CHUNK, CHUNK), pl.ds(E, E)],
            semo,
        ).start()

    def drain_out(b):
        pltpu.make_async_copy(
            rows_v.at[b],
            out_hbm.at[pl.ds(base, CHUNK), pl.ds(0, E)],
            semo,
        ).wait()
        pltpu.make_async_copy(
            ps_v.at[b],
            out_hbm.at[pl.ds(base, CHUNK), pl.ds(E, E)],
            semo,
        ).wait()

    fire(0, 0)

    def pair_body(k, carry):
        c0 = k * 2
        fire(c0 + 1, 1)
        drain_gathers(0)
        out_async(c0, 0)
        drain_gathers(1)
        drain_out(0)
        fire(c0 + 2, 0)
        out_async(c0 + 1, 1)
        drain_out(1)
        return carry

    lax.fori_loop(0, (N_CHUNKS - 1) // 2, pair_body, 0)
    drain_gathers(0)
    out_async(N_CHUNKS - 1, 0)
    drain_out(0)


SB = 8
BB = 512


def _ln_body(g_ref, gam_ref, bet_ref, out_ref):
    gam = gam_ref[...].reshape(1, 1, E)
    bet = bet_ref[...].reshape(1, 1, E)
    w = g_ref[...]
    e = w[:, :, :E] + w[:, :, E:]
    mean = jnp.mean(e, axis=-1, keepdims=True)
    d = e - mean
    var = jnp.mean(d * d, axis=-1, keepdims=True)
    normed = d * lax.rsqrt(var + 1e-5)
    res = normed * gam + bet
    for k in range(SB):
        out_ref[k, :, :] = res[:, k, :].T


def _tc_layernorm(gwide, gam, bet):
    return pl.pallas_call(
        _ln_body,
        grid=(S // SB, B // BB),
        in_specs=[
            pl.BlockSpec((BB, SB, 2 * E), lambda i, b: (b, i, 0)),
            pl.BlockSpec((1, E), lambda i, b: (0, 0)),
            pl.BlockSpec((1, E), lambda i, b: (0, 0)),
        ],
        out_specs=pl.BlockSpec((SB, E, BB), lambda i, b: (i, 0, b)),
        out_shape=jax.ShapeDtypeStruct((S, E, B), jnp.float32),
    )(gwide, gam, bet)


def kernel(x, segment_ids, token_table, pos_table, seg_table, ln_gamma, ln_beta):
    idx = x.reshape(NW, IDX_ROWS, IDX_W).astype(jnp.int32)
    ps_one = (pos_table[None, :S, :] + seg_table[:, None, :]).reshape(2 * S, E)
    ps_all = jnp.broadcast_to(ps_one[None], (NW, 2 * S, E)).reshape(NW * 2 * S, E)
    cidx = (segment_ids.astype(jnp.int32) * S
            + jnp.arange(S, dtype=jnp.int32)[None, :]).reshape(NW, IDX_ROWS, IDX_W)
    cidx = cidx + (jnp.arange(NW, dtype=jnp.int32) * (2 * S))[:, None, None]
    gathered = _sc_gather(token_table, ps_all, idx, cidx)
    out_t = _tc_layernorm(
        gathered.reshape(B, S, 2 * E),
        ln_gamma.reshape(1, E),
        ln_beta.reshape(1, E),
    )
    return jnp.transpose(out_t, (2, 0, 1))
